# baseline (speedup 1.0000x reference)
import functools

import jax
import jax.numpy as jnp
from jax.experimental import pallas as pl
from jax.experimental.pallas import tpu as pltpu


def _layer_body(a_ref, f_ref, w_ref, o_ref, b_ref, *, relu):
    @pl.when(pl.program_id(0) == 0)
    def _():
        f = f_ref[...]
        b_ref[0] = jnp.dot(
            f, w_ref[0], preferred_element_type=jnp.float32
        ).astype(jnp.bfloat16)
        b_ref[1] = jnp.dot(
            f, w_ref[1], preferred_element_type=jnp.float32
        ).astype(jnp.bfloat16)

    acc = jnp.dot(
        a_ref[0].astype(jnp.bfloat16), b_ref[0],
        preferred_element_type=jnp.float32,
    )
    acc = acc + jnp.dot(
        a_ref[1].astype(jnp.bfloat16), b_ref[1],
        preferred_element_type=jnp.float32,
    )
    if relu:
        acc = jnp.maximum(acc, 0.0)
    o_ref[...] = acc


def _poly_layer(poly_ls, feats, w, *, relu, block_m):
    p, n, _ = poly_ls.shape
    d_in = feats.shape[1]
    d_out = w.shape[2]
    grid = (n // block_m,)
    return pl.pallas_call(
        functools.partial(_layer_body, relu=relu),
        grid=grid,
        in_specs=[
            pl.BlockSpec((p, block_m, n), lambda i: (0, i, 0)),
            pl.BlockSpec((n, d_in), lambda i: (0, 0)),
            pl.BlockSpec((p, d_in, d_out), lambda i: (0, 0, 0)),
        ],
        out_specs=pl.BlockSpec((block_m, d_out), lambda i: (i, 0)),
        out_shape=jax.ShapeDtypeStruct((n, d_out), jnp.float32),
        scratch_shapes=[pltpu.VMEM((p, n, d_out), jnp.bfloat16)],
    )(poly_ls, feats, w)


def kernel(x, poly_ls, W0, W1):
    n = x.shape[0]
    block_m = 200 if n % 200 == 0 else 8
    h = _poly_layer(poly_ls, x, W0, relu=True, block_m=block_m)
    return _poly_layer(poly_ls, h, W1, relu=False, block_m=block_m)


# speedup vs baseline: 1.0037x; 1.0037x over previous
import functools

import jax
import jax.numpy as jnp
from jax.experimental import pallas as pl
from jax.experimental.pallas import tpu as pltpu


def _layer_body(a_ref, f_ref, w_ref, o_ref, b_ref, *, relu):
    @pl.when(pl.program_id(0) == 0)
    def _():
        f = f_ref[...]
        b_ref[0] = jnp.dot(f, w_ref[0], preferred_element_type=jnp.float32)
        b_ref[1] = jnp.dot(f, w_ref[1], preferred_element_type=jnp.float32)

    acc = jnp.dot(a_ref[0], b_ref[0], preferred_element_type=jnp.float32)
    acc = acc + jnp.dot(a_ref[1], b_ref[1], preferred_element_type=jnp.float32)
    if relu:
        acc = jnp.maximum(acc, 0.0)
    o_ref[...] = acc


def _poly_layer(poly_ls, feats, w, *, relu, block_m):
    p, n, _ = poly_ls.shape
    d_in = feats.shape[1]
    d_out = w.shape[2]
    grid = (n // block_m,)
    return pl.pallas_call(
        functools.partial(_layer_body, relu=relu),
        grid=grid,
        in_specs=[
            pl.BlockSpec((p, block_m, n), lambda i: (0, i, 0)),
            pl.BlockSpec((n, d_in), lambda i: (0, 0)),
            pl.BlockSpec((p, d_in, d_out), lambda i: (0, 0, 0)),
        ],
        out_specs=pl.BlockSpec((block_m, d_out), lambda i: (i, 0)),
        out_shape=jax.ShapeDtypeStruct((n, d_out), jnp.float32),
        scratch_shapes=[pltpu.VMEM((p, n, d_out), jnp.float32)],
    )(poly_ls, feats, w)


def kernel(x, poly_ls, W0, W1):
    n = x.shape[0]
    block_m = 200 if n % 200 == 0 else 8
    h = _poly_layer(poly_ls, x, W0, relu=True, block_m=block_m)
    return _poly_layer(poly_ls, h, W1, relu=False, block_m=block_m)


# speedup vs baseline: 1.0136x; 1.0099x over previous
import functools

import jax
import jax.numpy as jnp
from jax.experimental import pallas as pl
from jax.experimental.pallas import tpu as pltpu


def _body(a_ref, x_ref, w0_ref, w1_ref, o_ref, b0_ref, b1_ref, h_ref, *,
          nsteps, block_m):
    i = pl.program_id(0)

    @pl.when(i == 0)
    def _():
        x = x_ref[...]
        b0_ref[0] = jnp.dot(x, w0_ref[0], preferred_element_type=jnp.float32)
        b0_ref[1] = jnp.dot(x, w0_ref[1], preferred_element_type=jnp.float32)

    @pl.when(i < nsteps)
    def _():
        acc = jnp.dot(a_ref[0], b0_ref[0], preferred_element_type=jnp.float32)
        acc = acc + jnp.dot(a_ref[1], b0_ref[1],
                            preferred_element_type=jnp.float32)
        h_ref[pl.ds(i * block_m, block_m), :] = jnp.maximum(acc, 0.0).astype(
            jnp.bfloat16)

    @pl.when(i == nsteps)
    def _():
        h = h_ref[...]
        b1_ref[0] = jnp.dot(h, w1_ref[0].astype(jnp.bfloat16),
                            preferred_element_type=jnp.float32)
        b1_ref[1] = jnp.dot(h, w1_ref[1].astype(jnp.bfloat16),
                            preferred_element_type=jnp.float32)

    @pl.when(i >= nsteps)
    def _():
        acc = jnp.dot(a_ref[0], b1_ref[0], preferred_element_type=jnp.float32)
        acc = acc + jnp.dot(a_ref[1], b1_ref[1],
                            preferred_element_type=jnp.float32)
        o_ref[...] = acc


def kernel(x, poly_ls, W0, W1):
    p, n, _ = poly_ls.shape
    d_in = x.shape[1]
    d_h = W0.shape[2]
    d_out = W1.shape[2]
    block_m = 200 if n % 200 == 0 else 8
    nsteps = n // block_m
    return pl.pallas_call(
        functools.partial(_body, nsteps=nsteps, block_m=block_m),
        grid=(2 * nsteps,),
        in_specs=[
            pl.BlockSpec((p, block_m, n), lambda i: (0, i % nsteps, 0)),
            pl.BlockSpec((n, d_in), lambda i: (0, 0)),
            pl.BlockSpec((p, d_in, d_h), lambda i: (0, 0, 0)),
            pl.BlockSpec((p, d_h, d_out), lambda i: (0, 0, 0)),
        ],
        out_specs=pl.BlockSpec(
            (block_m, d_out),
            lambda i: (jnp.maximum(i - nsteps, 0), 0),
        ),
        out_shape=jax.ShapeDtypeStruct((n, d_out), jnp.float32),
        scratch_shapes=[
            pltpu.VMEM((p, n, d_h), jnp.float32),
            pltpu.VMEM((p, n, d_out), jnp.float32),
            pltpu.VMEM((n, d_h), jnp.bfloat16),
        ],
    )(poly_ls, x, W0, W1)


# speedup vs baseline: 1.0262x; 1.0125x over previous
import functools

import jax
import jax.numpy as jnp
from jax import lax
from jax.experimental import pallas as pl
from jax.experimental.pallas import tpu as pltpu


def _body(a_hbm, x_ref, w0_ref, w1_ref, o_ref,
          abuf, b0_ref, b1_ref, h_ref, sems, *, nsteps, block_m, nbuf):
    x = x_ref[...]
    b0_ref[0] = jnp.dot(x, w0_ref[0], preferred_element_type=jnp.float32)
    b0_ref[1] = jnp.dot(x, w0_ref[1], preferred_element_type=jnp.float32)

    total = 2 * nsteps

    def issue(step, slot):
        row = lax.rem(step, nsteps) * block_m
        pltpu.make_async_copy(
            a_hbm.at[:, pl.ds(row, block_m), :],
            abuf.at[slot],
            sems.at[slot],
        ).start()

    for s in range(nbuf - 1):
        issue(s, s)

    def step_fn(s, carry):
        @pl.when(s + nbuf - 1 < total)
        def _():
            issue(s + nbuf - 1, lax.rem(s + nbuf - 1, nbuf))

        slot = lax.rem(s, nbuf)
        pltpu.make_async_copy(
            a_hbm.at[:, pl.ds(0, block_m), :],
            abuf.at[slot],
            sems.at[slot],
        ).wait()

        row = lax.rem(s, nsteps) * block_m

        @pl.when(s < nsteps)
        def _():
            acc = jnp.dot(abuf[slot, 0], b0_ref[0],
                          preferred_element_type=jnp.float32)
            acc = acc + jnp.dot(abuf[slot, 1], b0_ref[1],
                                preferred_element_type=jnp.float32)
            h_ref[pl.ds(row, block_m), :] = jnp.maximum(acc, 0.0).astype(
                jnp.bfloat16)

        @pl.when(s == nsteps)
        def _():
            h = h_ref[...]
            b1_ref[0] = jnp.dot(h, w1_ref[0].astype(jnp.bfloat16),
                                preferred_element_type=jnp.float32)
            b1_ref[1] = jnp.dot(h, w1_ref[1].astype(jnp.bfloat16),
                                preferred_element_type=jnp.float32)

        @pl.when(s >= nsteps)
        def _():
            acc = jnp.dot(abuf[slot, 0], b1_ref[0],
                          preferred_element_type=jnp.float32)
            acc = acc + jnp.dot(abuf[slot, 1], b1_ref[1],
                                preferred_element_type=jnp.float32)
            o_ref[pl.ds(row, block_m), :] = acc

        return carry

    lax.fori_loop(0, total, step_fn, 0)


def kernel(x, poly_ls, W0, W1):
    p, n, _ = poly_ls.shape
    d_in = x.shape[1]
    d_h = W0.shape[2]
    d_out = W1.shape[2]
    block_m = 80 if n % 80 == 0 else 8
    nbuf = 4
    nsteps = n // block_m
    return pl.pallas_call(
        functools.partial(_body, nsteps=nsteps, block_m=block_m, nbuf=nbuf),
        in_specs=[
            pl.BlockSpec(memory_space=pl.ANY),
            pl.BlockSpec(memory_space=pltpu.VMEM),
            pl.BlockSpec(memory_space=pltpu.VMEM),
            pl.BlockSpec(memory_space=pltpu.VMEM),
        ],
        out_specs=pl.BlockSpec(memory_space=pltpu.VMEM),
        out_shape=jax.ShapeDtypeStruct((n, d_out), jnp.float32),
        scratch_shapes=[
            pltpu.VMEM((nbuf, p, block_m, n), jnp.float32),
            pltpu.VMEM((p, n, d_h), jnp.float32),
            pltpu.VMEM((p, n, d_out), jnp.float32),
            pltpu.VMEM((n, d_h), jnp.bfloat16),
            pltpu.SemaphoreType.DMA((nbuf,)),
        ],
    )(poly_ls, x, W0, W1)
